# Initial kernel scaffold; baseline (speedup 1.0000x reference)
import jax
import jax.numpy as jnp
from jax.experimental import pallas as pl


def kernel(feature, edge_index, W1, a1, W2, a2):
    raise NotImplementedError("write your pallas kernel here")


# speedup vs baseline: 12.8129x; 12.8129x over previous
import functools

import jax
import jax.numpy as jnp
from jax import lax
from jax.experimental import pallas as pl
from jax.experimental.pallas import tpu as pltpu
from jax.experimental.pallas import tpu_sc as plsc

N_NODES = 10000
D = 128
DH = 64
N_EDGES = 320000
EP = 327680
ROWS = EP // 128
VALID_ROWS = N_EDGES // 128
NC, NS = 2, 16
RB = ROWS // NS
NPAD = 10240
NN_T = NPAD // NS
LEAK = 0.2
EPS = 1e-9

_BLK = 1000


def _dense1_body(x_ref, w_ref, z_ref):
    z = jnp.dot(x_ref[...], w_ref[...], preferred_element_type=jnp.float32)
    z_ref[0] = z[:, :DH]
    z_ref[1] = z[:, DH:]


def _dense2_body(hp_ref, w_ref, z_ref):
    h = jnp.maximum(jnp.concatenate([hp_ref[0], hp_ref[1]], axis=1), 0.0)
    z = jnp.dot(h, w_ref[...], preferred_element_type=jnp.float32)
    z_ref[0] = z[:, :DH]
    z_ref[1] = z[:, DH:]


def _concat_body(hp_ref, o_ref):
    o_ref[...] = jnp.concatenate([hp_ref[0], hp_ref[1]], axis=1)


def _score_body(zh_ref, a8t_ref, s_ref, m_ref):
    st = (lax.dot_general(a8t_ref[:, :DH], zh_ref[0], (((1,), (1,)), ((), ())),
                          preferred_element_type=jnp.float32)
          + lax.dot_general(a8t_ref[:, DH:], zh_ref[1], (((1,), (1,)), ((), ())),
                            preferred_element_type=jnp.float32))
    s_ref[...] = st
    sM = jnp.max(st[0:1, :]) + jnp.max(st[1:2, :])
    M = jnp.where(sM > 0, sM, LEAK * sM)
    m_ref[...] = jnp.full((8, 128), M, jnp.float32)


def _dense1(x, W):
    return pl.pallas_call(
        _dense1_body,
        grid=(N_NODES // _BLK,),
        in_specs=[pl.BlockSpec((_BLK, D), lambda i: (i, 0)),
                  pl.BlockSpec((D, D), lambda i: (0, 0))],
        out_specs=pl.BlockSpec((2, _BLK, DH), lambda i: (0, i, 0)),
        out_shape=jax.ShapeDtypeStruct((2, N_NODES, DH), jnp.float32),
    )(x, W)


def _dense2(hp, W):
    return pl.pallas_call(
        _dense2_body,
        grid=(N_NODES // _BLK,),
        in_specs=[pl.BlockSpec((2, _BLK, DH), lambda i: (0, i, 0)),
                  pl.BlockSpec((D, D), lambda i: (0, 0))],
        out_specs=pl.BlockSpec((2, _BLK, DH), lambda i: (0, i, 0)),
        out_shape=jax.ShapeDtypeStruct((2, N_NODES, DH), jnp.float32),
    )(hp, W)


def _concat(hp):
    return pl.pallas_call(
        _concat_body,
        grid=(N_NODES // _BLK,),
        in_specs=[pl.BlockSpec((2, _BLK, DH), lambda i: (0, i, 0))],
        out_specs=pl.BlockSpec((_BLK, D), lambda i: (i, 0)),
        out_shape=jax.ShapeDtypeStruct((N_NODES, D), jnp.float32),
    )(hp)


def _score(zh, a8t):
    return pl.pallas_call(
        _score_body,
        in_specs=[pl.BlockSpec((2, N_NODES, DH), lambda: (0, 0, 0)),
                  pl.BlockSpec((8, D), lambda: (0, 0))],
        out_specs=[pl.BlockSpec((8, N_NODES), lambda: (0, 0)),
                   pl.BlockSpec((8, 128), lambda: (0, 0))],
        out_shape=[jax.ShapeDtypeStruct((8, N_NODES), jnp.float32),
                   jax.ShapeDtypeStruct((8, 128), jnp.float32)],
    )(zh, a8t)


_MESH = plsc.VectorSubcoreMesh(core_axis_name="c", subcore_axis_name="s",
                               num_cores=NC, num_subcores=NS)


@functools.partial(
    pl.kernel,
    mesh=_MESH,
    out_type=jax.ShapeDtypeStruct((NC, N_NODES, DH), jnp.float32),
    compiler_params=pltpu.CompilerParams(needs_layout_passes=False,
                                         use_tc_tiling_on_sc=False),
    scratch_types=[
        pltpu.VMEM((N_NODES,), jnp.float32),
        pltpu.VMEM((N_NODES,), jnp.float32),
        pltpu.VMEM((3, 128), jnp.int32),
        pltpu.VMEM((3, 128), jnp.int32),
        pltpu.VMEM((2, 128), jnp.float32),
        pltpu.VMEM((2, 128, DH), jnp.float32),
        pltpu.VMEM((80,), jnp.float32),
        pltpu.VMEM((NN_T,), jnp.float32),
        pltpu.VMEM((16,), jnp.float32),
        pltpu.VMEM_SHARED((NPAD, DH), jnp.float32),
        pltpu.VMEM_SHARED((NPAD,), jnp.float32),
        pltpu.SemaphoreType.DMA,
        pltpu.SemaphoreType.DMA,
        pltpu.SemaphoreType.DMA,
    ],
)
def _sc_layer(zl_hbm, zr_hbm, s8t_hbm, m_hbm, src_hbm, dst_hbm, hpart_hbm,
              s1v, s2v, srcd, dstd, exr, zbuf, dvm, dz, mv,
              acc_sh, den_sh, sem_g, sem_i, sem_d):
    c = lax.axis_index("c")
    s = lax.axis_index("s")

    pltpu.sync_copy(s8t_hbm.at[0], s1v)
    pltpu.sync_copy(s8t_hbm.at[1], s2v)
    pltpu.sync_copy(m_hbm.at[0, pl.ds(0, 16)], mv)

    def _zrow(r, _):
        for k in range(DH // 16):
            zbuf[0, r, pl.ds(k * 16, 16)] = jnp.zeros((16,), jnp.float32)
        return 0
    lax.fori_loop(0, 128, _zrow, 0)

    def _zacc(k, _):
        pltpu.sync_copy(zbuf.at[0], acc_sh.at[pl.ds(s * NN_T + k * 128, 128)])
        return 0
    lax.fori_loop(0, NN_T // 128, _zacc, 0)

    def _zden(i, _):
        dz[pl.ds(i * 16, 16)] = jnp.zeros((16,), jnp.float32)
        return 0
    lax.fori_loop(0, NN_T // 16, _zden, 0)
    pltpu.sync_copy(dz, den_sh.at[pl.ds(s * NN_T, NN_T)])

    plsc.subcore_barrier()

    M = mv[...][0]
    row0 = s * RB

    def _fire_idx(j, slot):
        pltpu.async_copy(src_hbm.at[row0 + j], srcd.at[slot], sem_i)
        pltpu.async_copy(dst_hbm.at[row0 + j], dstd.at[slot], sem_i)

    def _wait_idx():
        pltpu.make_async_copy(src_hbm.at[0], srcd.at[0], sem_i).wait()
        pltpu.make_async_copy(src_hbm.at[0], srcd.at[0], sem_i).wait()

    def _edge_loop(ztab_hbm):
        _fire_idx(0, 0)
        _fire_idx(1, 1)
        _wait_idx()
        pltpu.async_copy(ztab_hbm.at[srcd.at[0]], zbuf.at[0], sem_g)

        def _body(j, _):
            buf = j % 2
            islot = j % 3
            eslot = j % 2

            @pl.when(j < RB - 1)
            def _():
                _wait_idx()
                pltpu.async_copy(ztab_hbm.at[srcd.at[(j + 1) % 3]],
                                 zbuf.at[(j + 1) % 2], sem_g)

            @pl.when(j < RB - 2)
            def _():
                _fire_idx(j + 2, (j + 2) % 3)

            pltpu.make_async_copy(ztab_hbm.at[srcd.at[0]], zbuf.at[0],
                                  sem_g).wait()

            valid = row0 + j < VALID_ROWS

            def _scale(g, _):
                sv = srcd[islot, pl.ds(g * 16, 16)]
                dv = dstd[islot, pl.ds(g * 16, 16)]
                e = plsc.load_gather(s1v, [sv]) + plsc.load_gather(s2v, [dv])
                e = jnp.where(e > 0, e, LEAK * e) - M
                ex = jnp.exp(e)
                ex = jnp.where(valid, ex, jnp.zeros((16,), jnp.float32))
                exr[eslot, pl.ds(g * 16, 16)] = ex
                for t in range(16):
                    r = g * 16 + t
                    ex_s = ex[t]
                    for k in range(DH // 16):
                        zbuf[buf, r, pl.ds(k * 16, 16)] = (
                            zbuf[buf, r, pl.ds(k * 16, 16)] * ex_s)
                return 0
            lax.fori_loop(0, 8, _scale, 0)

            pltpu.sync_copy(exr.at[eslot], den_sh.at[dstd.at[islot]], add=True)
            pltpu.sync_copy(zbuf.at[buf], acc_sh.at[dstd.at[islot]], add=True)
            return 0
        lax.fori_loop(0, RB, _body, 0)

    @pl.when(c == 0)
    def _():
        _edge_loop(zl_hbm)

    @pl.when(c == 1)
    def _():
        _edge_loop(zr_hbm)

    plsc.subcore_barrier()

    base0 = s * NN_T
    nch = jnp.where(s == NS - 1, (N_NODES - (NS - 1) * NN_T) // 80,
                    NN_T // 80)

    def _nbody(k, _):
        base = base0 + k * 80
        pltpu.sync_copy(acc_sh.at[pl.ds(base, 80)], zbuf.at[0, pl.ds(0, 80)])
        pltpu.sync_copy(den_sh.at[pl.ds(base, 80)], dvm)

        def _gbody(g, _):
            inv = 1.0 / (dvm[pl.ds(g * 16, 16)] + EPS)
            for t in range(16):
                r = g * 16 + t
                inv_s = inv[t]
                for k2 in range(DH // 16):
                    zbuf[0, r, pl.ds(k2 * 16, 16)] = (
                        zbuf[0, r, pl.ds(k2 * 16, 16)] * inv_s)
            return 0
        lax.fori_loop(0, 5, _gbody, 0)

        pltpu.sync_copy(zbuf.at[0, pl.ds(0, 80)],
                        hpart_hbm.at[c, pl.ds(base, 80)])
        return 0
    lax.fori_loop(0, nch, _nbody, 0)


def _att_vec(a):
    a8 = jnp.zeros((8, D), jnp.float32)
    a8 = a8.at[0].set(a[:D]).at[1].set(a[D:])
    return a8


def kernel(feature, edge_index, W1, a1, W2, a2):
    src = edge_index[0].astype(jnp.int32)
    dst = edge_index[1].astype(jnp.int32)
    src2d = jnp.pad(src, (0, EP - N_EDGES)).reshape(ROWS, 128)
    dst2d = jnp.pad(dst, (0, EP - N_EDGES)).reshape(ROWS, 128)

    zh1 = _dense1(feature, W1)
    s8t1, m1 = _score(zh1, _att_vec(a1))
    hp1 = _sc_layer(zh1[0], zh1[1], s8t1, m1, src2d, dst2d)
    zh2 = _dense2(hp1, W2)
    s8t2, m2 = _score(zh2, _att_vec(a2))
    hp2 = _sc_layer(zh2[0], zh2[1], s8t2, m2, src2d, dst2d)
    return _concat(hp2)


# speedup vs baseline: 12.9373x; 1.0097x over previous
import functools

import jax
import jax.numpy as jnp
from jax import lax
from jax.experimental import pallas as pl
from jax.experimental.pallas import tpu as pltpu
from jax.experimental.pallas import tpu_sc as plsc

N_NODES = 10000
D = 128
DH = 64
N_EDGES = 320000
EP = 327680
ROWS = EP // 128
VALID_ROWS = N_EDGES // 128
NC, NS = 2, 16
RB = ROWS // NS
NPAD = 10240
NN_T = NPAD // NS
LEAK = 0.2
EPS = 1e-9

_BLK = 1000


def _dense1_body(x_ref, w_ref, z_ref):
    z = jnp.dot(x_ref[...], w_ref[...], preferred_element_type=jnp.float32)
    z_ref[0] = z[:, :DH]
    z_ref[1] = z[:, DH:]


def _dense2_body(hp_ref, w_ref, z_ref):
    h = jnp.maximum(jnp.concatenate([hp_ref[0], hp_ref[1]], axis=1), 0.0)
    z = jnp.dot(h, w_ref[...], preferred_element_type=jnp.float32)
    z_ref[0] = z[:, :DH]
    z_ref[1] = z[:, DH:]


def _concat_body(hp_ref, o_ref):
    o_ref[...] = jnp.concatenate([hp_ref[0], hp_ref[1]], axis=1)


def _score_body(zh_ref, a8t_ref, s_ref, m_ref):
    st = (lax.dot_general(a8t_ref[:, :DH], zh_ref[0], (((1,), (1,)), ((), ())),
                          preferred_element_type=jnp.float32)
          + lax.dot_general(a8t_ref[:, DH:], zh_ref[1], (((1,), (1,)), ((), ())),
                            preferred_element_type=jnp.float32))
    s_ref[...] = st
    sM = jnp.max(st[0:1, :]) + jnp.max(st[1:2, :])
    M = jnp.where(sM > 0, sM, LEAK * sM)
    m_ref[...] = jnp.full((8, 128), M, jnp.float32)


def _dense1(x, W):
    return pl.pallas_call(
        _dense1_body,
        grid=(N_NODES // _BLK,),
        in_specs=[pl.BlockSpec((_BLK, D), lambda i: (i, 0)),
                  pl.BlockSpec((D, D), lambda i: (0, 0))],
        out_specs=pl.BlockSpec((2, _BLK, DH), lambda i: (0, i, 0)),
        out_shape=jax.ShapeDtypeStruct((2, N_NODES, DH), jnp.float32),
    )(x, W)


def _dense2(hp, W):
    return pl.pallas_call(
        _dense2_body,
        grid=(N_NODES // _BLK,),
        in_specs=[pl.BlockSpec((2, _BLK, DH), lambda i: (0, i, 0)),
                  pl.BlockSpec((D, D), lambda i: (0, 0))],
        out_specs=pl.BlockSpec((2, _BLK, DH), lambda i: (0, i, 0)),
        out_shape=jax.ShapeDtypeStruct((2, N_NODES, DH), jnp.float32),
    )(hp, W)


def _concat(hp):
    return pl.pallas_call(
        _concat_body,
        grid=(N_NODES // _BLK,),
        in_specs=[pl.BlockSpec((2, _BLK, DH), lambda i: (0, i, 0))],
        out_specs=pl.BlockSpec((_BLK, D), lambda i: (i, 0)),
        out_shape=jax.ShapeDtypeStruct((N_NODES, D), jnp.float32),
    )(hp)


def _score(zh, a8t):
    return pl.pallas_call(
        _score_body,
        in_specs=[pl.BlockSpec((2, N_NODES, DH), lambda: (0, 0, 0)),
                  pl.BlockSpec((8, D), lambda: (0, 0))],
        out_specs=[pl.BlockSpec((8, N_NODES), lambda: (0, 0)),
                   pl.BlockSpec((8, 128), lambda: (0, 0))],
        out_shape=[jax.ShapeDtypeStruct((8, N_NODES), jnp.float32),
                   jax.ShapeDtypeStruct((8, 128), jnp.float32)],
    )(zh, a8t)


_MESH = plsc.VectorSubcoreMesh(core_axis_name="c", subcore_axis_name="s",
                               num_cores=NC, num_subcores=NS)


@functools.partial(
    pl.kernel,
    mesh=_MESH,
    out_type=jax.ShapeDtypeStruct((NC, N_NODES, DH), jnp.float32),
    compiler_params=pltpu.CompilerParams(needs_layout_passes=False,
                                         use_tc_tiling_on_sc=False),
    scratch_types=[
        pltpu.VMEM((N_NODES,), jnp.float32),
        pltpu.VMEM((N_NODES,), jnp.float32),
        pltpu.VMEM((3, 128), jnp.int32),
        pltpu.VMEM((3, 128), jnp.int32),
        pltpu.VMEM((2, 128), jnp.float32),
        pltpu.VMEM((2, 128, DH), jnp.float32),
        pltpu.VMEM((80,), jnp.float32),
        pltpu.VMEM((NN_T,), jnp.float32),
        pltpu.VMEM((16,), jnp.float32),
        pltpu.VMEM_SHARED((NPAD, DH), jnp.float32),
        pltpu.VMEM_SHARED((NPAD,), jnp.float32),
        pltpu.SemaphoreType.DMA,
        pltpu.SemaphoreType.DMA,
        pltpu.SemaphoreType.DMA,
        pltpu.SemaphoreType.DMA,
    ],
)
def _sc_layer(zl_hbm, zr_hbm, s8t_hbm, m_hbm, src_hbm, dst_hbm, hpart_hbm,
              s1v, s2v, srcd, dstd, exr, zbuf, dvm, dz, mv,
              acc_sh, den_sh, sem_g, sem_i, sem_d, sem_s):
    c = lax.axis_index("c")
    s = lax.axis_index("s")

    pltpu.sync_copy(s8t_hbm.at[0], s1v)
    pltpu.sync_copy(s8t_hbm.at[1], s2v)
    pltpu.sync_copy(m_hbm.at[0, pl.ds(0, 16)], mv)

    def _zrow(r, _):
        for k in range(DH // 16):
            zbuf[0, r, pl.ds(k * 16, 16)] = jnp.zeros((16,), jnp.float32)
        return 0
    lax.fori_loop(0, 128, _zrow, 0)

    def _zacc(k, _):
        pltpu.sync_copy(zbuf.at[0], acc_sh.at[pl.ds(s * NN_T + k * 128, 128)])
        return 0
    lax.fori_loop(0, NN_T // 128, _zacc, 0)

    def _zden(i, _):
        dz[pl.ds(i * 16, 16)] = jnp.zeros((16,), jnp.float32)
        return 0
    lax.fori_loop(0, NN_T // 16, _zden, 0)
    pltpu.sync_copy(dz, den_sh.at[pl.ds(s * NN_T, NN_T)])

    plsc.subcore_barrier()

    M = mv[...][0]
    row0 = s * RB

    def _fire_idx(j, slot):
        pltpu.async_copy(src_hbm.at[row0 + j], srcd.at[slot], sem_i)
        pltpu.async_copy(dst_hbm.at[row0 + j], dstd.at[slot], sem_i)

    def _wait_idx():
        pltpu.make_async_copy(src_hbm.at[0], srcd.at[0], sem_i).wait()
        pltpu.make_async_copy(src_hbm.at[0], srcd.at[0], sem_i).wait()

    def _edge_loop(ztab_hbm):
        _fire_idx(0, 0)
        _fire_idx(1, 1)
        _wait_idx()
        pltpu.async_copy(ztab_hbm.at[srcd.at[0]], zbuf.at[0], sem_g)

        def _body(j, _):
            buf = j % 2
            islot = j % 3
            eslot = j % 2

            @pl.when(j >= 1)
            def _():
                pltpu.make_async_copy(zbuf.at[0], acc_sh.at[dstd.at[0]],
                                      sem_s).wait()
                pltpu.make_async_copy(exr.at[0], den_sh.at[dstd.at[0]],
                                      sem_d).wait()

            @pl.when(j < RB - 1)
            def _():
                _wait_idx()
                pltpu.async_copy(ztab_hbm.at[srcd.at[(j + 1) % 3]],
                                 zbuf.at[(j + 1) % 2], sem_g)

            @pl.when(j < RB - 2)
            def _():
                _fire_idx(j + 2, (j + 2) % 3)

            pltpu.make_async_copy(ztab_hbm.at[srcd.at[0]], zbuf.at[0],
                                  sem_g).wait()

            valid = row0 + j < VALID_ROWS

            def _scale(g, _):
                sv = srcd[islot, pl.ds(g * 16, 16)]
                dv = dstd[islot, pl.ds(g * 16, 16)]
                e = plsc.load_gather(s1v, [sv]) + plsc.load_gather(s2v, [dv])
                e = jnp.where(e > 0, e, LEAK * e) - M
                ex = jnp.exp(e)
                ex = jnp.where(valid, ex, jnp.zeros((16,), jnp.float32))
                exr[eslot, pl.ds(g * 16, 16)] = ex
                for t in range(16):
                    r = g * 16 + t
                    ex_s = ex[t]
                    for k in range(DH // 16):
                        zbuf[buf, r, pl.ds(k * 16, 16)] = (
                            zbuf[buf, r, pl.ds(k * 16, 16)] * ex_s)
                return 0
            lax.fori_loop(0, 8, _scale, 0)

            pltpu.async_copy(exr.at[eslot], den_sh.at[dstd.at[islot]], sem_d,
                             add=True)
            pltpu.async_copy(zbuf.at[buf], acc_sh.at[dstd.at[islot]], sem_s,
                             add=True)
            return 0
        lax.fori_loop(0, RB, _body, 0)
        pltpu.make_async_copy(zbuf.at[0], acc_sh.at[dstd.at[0]], sem_s).wait()
        pltpu.make_async_copy(exr.at[0], den_sh.at[dstd.at[0]], sem_d).wait()

    @pl.when(c == 0)
    def _():
        _edge_loop(zl_hbm)

    @pl.when(c == 1)
    def _():
        _edge_loop(zr_hbm)

    plsc.subcore_barrier()

    base0 = s * NN_T
    nch = jnp.where(s == NS - 1, (N_NODES - (NS - 1) * NN_T) // 80,
                    NN_T // 80)

    def _nbody(k, _):
        base = base0 + k * 80
        pltpu.sync_copy(acc_sh.at[pl.ds(base, 80)], zbuf.at[0, pl.ds(0, 80)])
        pltpu.sync_copy(den_sh.at[pl.ds(base, 80)], dvm)

        def _gbody(g, _):
            inv = 1.0 / (dvm[pl.ds(g * 16, 16)] + EPS)
            for t in range(16):
                r = g * 16 + t
                inv_s = inv[t]
                for k2 in range(DH // 16):
                    zbuf[0, r, pl.ds(k2 * 16, 16)] = (
                        zbuf[0, r, pl.ds(k2 * 16, 16)] * inv_s)
            return 0
        lax.fori_loop(0, 5, _gbody, 0)

        pltpu.sync_copy(zbuf.at[0, pl.ds(0, 80)],
                        hpart_hbm.at[c, pl.ds(base, 80)])
        return 0
    lax.fori_loop(0, nch, _nbody, 0)


def _att_vec(a):
    a8 = jnp.zeros((8, D), jnp.float32)
    a8 = a8.at[0].set(a[:D]).at[1].set(a[D:])
    return a8


def kernel(feature, edge_index, W1, a1, W2, a2):
    src = edge_index[0].astype(jnp.int32)
    dst = edge_index[1].astype(jnp.int32)
    src2d = jnp.pad(src, (0, EP - N_EDGES)).reshape(ROWS, 128)
    dst2d = jnp.pad(dst, (0, EP - N_EDGES)).reshape(ROWS, 128)

    zh1 = _dense1(feature, W1)
    s8t1, m1 = _score(zh1, _att_vec(a1))
    hp1 = _sc_layer(zh1[0], zh1[1], s8t1, m1, src2d, dst2d)
    zh2 = _dense2(hp1, W2)
    s8t2, m2 = _score(zh2, _att_vec(a2))
    hp2 = _sc_layer(zh2[0], zh2[1], s8t2, m2, src2d, dst2d)
    return _concat(hp2)


# speedup vs baseline: 17.8959x; 1.3833x over previous
import functools

import jax
import jax.numpy as jnp
from jax import lax
from jax.experimental import pallas as pl
from jax.experimental.pallas import tpu as pltpu
from jax.experimental.pallas import tpu_sc as plsc

N_NODES = 10000
D = 128
DH = 64
N_EDGES = 320000
EP = 327680
ROWS = EP // 128
VALID_ROWS = N_EDGES // 128
NC, NS = 2, 16
RB = ROWS // NS
NPAD = 10240
NN_T = NPAD // NS
LEAK = 0.2
EPS = 1e-9

_BLK = 1000


def _dense1_body(x_ref, w_ref, z_ref):
    z = jnp.dot(x_ref[...], w_ref[...], preferred_element_type=jnp.float32)
    z_ref[0] = z[:, :DH]
    z_ref[1] = z[:, DH:]


def _dense2_body(hp_ref, w_ref, z_ref):
    h = jnp.maximum(jnp.concatenate([hp_ref[0], hp_ref[1]], axis=1), 0.0)
    z = jnp.dot(h, w_ref[...], preferred_element_type=jnp.float32)
    z_ref[0] = z[:, :DH]
    z_ref[1] = z[:, DH:]


def _concat_body(hp_ref, o_ref):
    o_ref[...] = jnp.concatenate([hp_ref[0], hp_ref[1]], axis=1)


def _score_body(zh_ref, a8t_ref, s_ref, m_ref):
    st = (lax.dot_general(a8t_ref[:, :DH], zh_ref[0], (((1,), (1,)), ((), ())),
                          preferred_element_type=jnp.float32)
          + lax.dot_general(a8t_ref[:, DH:], zh_ref[1], (((1,), (1,)), ((), ())),
                            preferred_element_type=jnp.float32))
    s_ref[...] = st
    sM = jnp.max(st[0:1, :]) + jnp.max(st[1:2, :])
    M = jnp.where(sM > 0, sM, LEAK * sM)
    m_ref[...] = jnp.full((8, 128), M, jnp.float32)


def _dense1(x, W):
    return pl.pallas_call(
        _dense1_body,
        grid=(N_NODES // _BLK,),
        in_specs=[pl.BlockSpec((_BLK, D), lambda i: (i, 0)),
                  pl.BlockSpec((D, D), lambda i: (0, 0))],
        out_specs=pl.BlockSpec((2, _BLK, DH), lambda i: (0, i, 0)),
        out_shape=jax.ShapeDtypeStruct((2, N_NODES, DH), jnp.float32),
    )(x, W)


def _dense2(hp, W):
    return pl.pallas_call(
        _dense2_body,
        grid=(N_NODES // _BLK,),
        in_specs=[pl.BlockSpec((2, _BLK, DH), lambda i: (0, i, 0)),
                  pl.BlockSpec((D, D), lambda i: (0, 0))],
        out_specs=pl.BlockSpec((2, _BLK, DH), lambda i: (0, i, 0)),
        out_shape=jax.ShapeDtypeStruct((2, N_NODES, DH), jnp.float32),
    )(hp, W)


def _concat(hp):
    return pl.pallas_call(
        _concat_body,
        grid=(N_NODES // _BLK,),
        in_specs=[pl.BlockSpec((2, _BLK, DH), lambda i: (0, i, 0))],
        out_specs=pl.BlockSpec((_BLK, D), lambda i: (i, 0)),
        out_shape=jax.ShapeDtypeStruct((N_NODES, D), jnp.float32),
    )(hp)


def _score(zh, a8t):
    return pl.pallas_call(
        _score_body,
        in_specs=[pl.BlockSpec((2, N_NODES, DH), lambda: (0, 0, 0)),
                  pl.BlockSpec((8, D), lambda: (0, 0))],
        out_specs=[pl.BlockSpec((8, N_NODES), lambda: (0, 0)),
                   pl.BlockSpec((8, 128), lambda: (0, 0))],
        out_shape=[jax.ShapeDtypeStruct((8, N_NODES), jnp.float32),
                   jax.ShapeDtypeStruct((8, 128), jnp.float32)],
    )(zh, a8t)


_MESH = plsc.VectorSubcoreMesh(core_axis_name="c", subcore_axis_name="s",
                               num_cores=NC, num_subcores=NS)


@functools.partial(
    pl.kernel,
    mesh=_MESH,
    out_type=jax.ShapeDtypeStruct((NC, N_NODES, DH), jnp.float32),
    compiler_params=pltpu.CompilerParams(needs_layout_passes=False,
                                         use_tc_tiling_on_sc=False),
    scratch_types=[
        pltpu.VMEM((N_NODES,), jnp.float32),
        pltpu.VMEM((N_NODES,), jnp.float32),
        pltpu.VMEM((3, 128), jnp.int32),
        pltpu.VMEM((3, 128), jnp.int32),
        pltpu.VMEM((2, 128), jnp.float32),
        pltpu.VMEM((2, 128, DH), jnp.float32),
        pltpu.VMEM((80,), jnp.float32),
        pltpu.VMEM((NN_T,), jnp.float32),
        pltpu.VMEM((16,), jnp.float32),
        pltpu.VMEM_SHARED((NPAD, DH), jnp.float32),
        pltpu.VMEM_SHARED((NPAD,), jnp.float32),
        pltpu.SemaphoreType.DMA,
        pltpu.SemaphoreType.DMA,
        pltpu.SemaphoreType.DMA,
        pltpu.SemaphoreType.DMA,
    ],
)
def _sc_layer(zl_hbm, zr_hbm, s8t_hbm, m_hbm, src_hbm, dst_hbm, hpart_hbm,
              s1v, s2v, srcd, dstd, exr, zbuf, dvm, dz, mv,
              acc_sh, den_sh, sem_g, sem_i, sem_d, sem_s):
    c = lax.axis_index("c")
    s = lax.axis_index("s")

    pltpu.sync_copy(s8t_hbm.at[0], s1v)
    pltpu.sync_copy(s8t_hbm.at[1], s2v)
    pltpu.sync_copy(m_hbm.at[0, pl.ds(0, 16)], mv)

    def _zrow(r, _):
        for k in range(DH // 16):
            zbuf[0, r, pl.ds(k * 16, 16)] = jnp.zeros((16,), jnp.float32)
        return 0
    lax.fori_loop(0, 128, _zrow, 0)

    def _zacc(k, _):
        pltpu.sync_copy(zbuf.at[0], acc_sh.at[pl.ds(s * NN_T + k * 128, 128)])
        return 0
    lax.fori_loop(0, NN_T // 128, _zacc, 0)

    def _zden(i, _):
        dz[pl.ds(i * 16, 16)] = jnp.zeros((16,), jnp.float32)
        return 0
    lax.fori_loop(0, NN_T // 16, _zden, 0)
    pltpu.sync_copy(dz, den_sh.at[pl.ds(s * NN_T, NN_T)])

    plsc.subcore_barrier()

    M = mv[...][0]
    row0 = s * RB

    def _fire_idx(j, slot):
        pltpu.async_copy(src_hbm.at[row0 + j], srcd.at[slot], sem_i)
        pltpu.async_copy(dst_hbm.at[row0 + j], dstd.at[slot], sem_i)

    def _wait_idx():
        pltpu.make_async_copy(src_hbm.at[0], srcd.at[0], sem_i).wait()
        pltpu.make_async_copy(src_hbm.at[0], srcd.at[0], sem_i).wait()

    def _edge_loop(ztab_hbm):
        _fire_idx(0, 0)
        _fire_idx(1, 1)
        _wait_idx()
        pltpu.async_copy(ztab_hbm.at[srcd.at[0]], zbuf.at[0], sem_g)

        def _body(j, _):
            buf = j % 2
            islot = j % 3
            eslot = j % 2

            @pl.when(j >= 1)
            def _():
                pltpu.make_async_copy(zbuf.at[0], acc_sh.at[dstd.at[0]],
                                      sem_s).wait()
                pltpu.make_async_copy(exr.at[0], den_sh.at[dstd.at[0]],
                                      sem_d).wait()

            @pl.when(j < RB - 1)
            def _():
                _wait_idx()
                pltpu.async_copy(ztab_hbm.at[srcd.at[(j + 1) % 3]],
                                 zbuf.at[(j + 1) % 2], sem_g)

            @pl.when(j < RB - 2)
            def _():
                _fire_idx(j + 2, (j + 2) % 3)

            pltpu.make_async_copy(ztab_hbm.at[srcd.at[0]], zbuf.at[0],
                                  sem_g).wait()

            valid = row0 + j < VALID_ROWS

            for g in range(8):
                sv = srcd[islot, pl.ds(g * 16, 16)]
                dv = dstd[islot, pl.ds(g * 16, 16)]
                e = plsc.load_gather(s1v, [sv]) + plsc.load_gather(s2v, [dv])
                e = jnp.where(e > 0, e, LEAK * e) - M
                ex = jnp.exp(e)
                ex = jnp.where(valid, ex, jnp.zeros((16,), jnp.float32))
                exr[eslot, pl.ds(g * 16, 16)] = ex
                for t in range(16):
                    r = g * 16 + t
                    ex_s = ex[t]
                    for k in range(DH // 16):
                        zbuf[buf, r, pl.ds(k * 16, 16)] = (
                            zbuf[buf, r, pl.ds(k * 16, 16)] * ex_s)

            pltpu.async_copy(exr.at[eslot], den_sh.at[dstd.at[islot]], sem_d,
                             add=True)
            pltpu.async_copy(zbuf.at[buf], acc_sh.at[dstd.at[islot]], sem_s,
                             add=True)
            return 0
        lax.fori_loop(0, RB, _body, 0)
        pltpu.make_async_copy(zbuf.at[0], acc_sh.at[dstd.at[0]], sem_s).wait()
        pltpu.make_async_copy(exr.at[0], den_sh.at[dstd.at[0]], sem_d).wait()

    @pl.when(c == 0)
    def _():
        _edge_loop(zl_hbm)

    @pl.when(c == 1)
    def _():
        _edge_loop(zr_hbm)

    plsc.subcore_barrier()

    base0 = s * NN_T
    nch = jnp.where(s == NS - 1, (N_NODES - (NS - 1) * NN_T) // 80,
                    NN_T // 80)

    def _nbody(k, _):
        base = base0 + k * 80
        pltpu.sync_copy(acc_sh.at[pl.ds(base, 80)], zbuf.at[0, pl.ds(0, 80)])
        pltpu.sync_copy(den_sh.at[pl.ds(base, 80)], dvm)

        def _gbody(g, _):
            inv = 1.0 / (dvm[pl.ds(g * 16, 16)] + EPS)
            for t in range(16):
                r = g * 16 + t
                inv_s = inv[t]
                for k2 in range(DH // 16):
                    zbuf[0, r, pl.ds(k2 * 16, 16)] = (
                        zbuf[0, r, pl.ds(k2 * 16, 16)] * inv_s)
            return 0
        lax.fori_loop(0, 5, _gbody, 0)

        pltpu.sync_copy(zbuf.at[0, pl.ds(0, 80)],
                        hpart_hbm.at[c, pl.ds(base, 80)])
        return 0
    lax.fori_loop(0, nch, _nbody, 0)


def _att_vec(a):
    a8 = jnp.zeros((8, D), jnp.float32)
    a8 = a8.at[0].set(a[:D]).at[1].set(a[D:])
    return a8


def kernel(feature, edge_index, W1, a1, W2, a2):
    src = edge_index[0].astype(jnp.int32)
    dst = edge_index[1].astype(jnp.int32)
    src2d = jnp.pad(src, (0, EP - N_EDGES)).reshape(ROWS, 128)
    dst2d = jnp.pad(dst, (0, EP - N_EDGES)).reshape(ROWS, 128)

    zh1 = _dense1(feature, W1)
    s8t1, m1 = _score(zh1, _att_vec(a1))
    hp1 = _sc_layer(zh1[0], zh1[1], s8t1, m1, src2d, dst2d)
    zh2 = _dense2(hp1, W2)
    s8t2, m2 = _score(zh2, _att_vec(a2))
    hp2 = _sc_layer(zh2[0], zh2[1], s8t2, m2, src2d, dst2d)
    return _concat(hp2)


# speedup vs baseline: 19.0180x; 1.0627x over previous
import functools

import jax
import jax.numpy as jnp
from jax import lax
from jax.experimental import pallas as pl
from jax.experimental.pallas import tpu as pltpu
from jax.experimental.pallas import tpu_sc as plsc

N_NODES = 10000
D = 128
DH = 64
N_EDGES = 320000
EP = 327680
ROWS = EP // 128
VALID_ROWS = N_EDGES // 128
NC, NS = 2, 16
RB = ROWS // NS
NPAD = 10240
NN_T = NPAD // NS
LEAK = 0.2
EPS = 1e-9

_BLK = 1000


def _dense1_body(x_ref, w_ref, z_ref):
    z = jnp.dot(x_ref[...], w_ref[...], preferred_element_type=jnp.float32)
    z_ref[0] = z[:, :DH]
    z_ref[1] = z[:, DH:]


def _dense2_body(hp_ref, w_ref, z_ref):
    h = jnp.maximum(jnp.concatenate([hp_ref[0], hp_ref[1]], axis=1), 0.0)
    z = jnp.dot(h, w_ref[...], preferred_element_type=jnp.float32)
    z_ref[0] = z[:, :DH]
    z_ref[1] = z[:, DH:]


def _concat_body(hp_ref, o_ref):
    o_ref[...] = jnp.concatenate([hp_ref[0], hp_ref[1]], axis=1)


def _score_body(zh_ref, a8t_ref, s_ref, m_ref):
    st = (lax.dot_general(a8t_ref[:, :DH], zh_ref[0], (((1,), (1,)), ((), ())),
                          preferred_element_type=jnp.float32)
          + lax.dot_general(a8t_ref[:, DH:], zh_ref[1], (((1,), (1,)), ((), ())),
                            preferred_element_type=jnp.float32))
    s_ref[...] = st
    sM = jnp.max(st[0:1, :]) + jnp.max(st[1:2, :])
    M = jnp.where(sM > 0, sM, LEAK * sM)
    m_ref[...] = jnp.full((8, 128), M, jnp.float32)


def _dense1(x, W):
    return pl.pallas_call(
        _dense1_body,
        grid=(N_NODES // _BLK,),
        in_specs=[pl.BlockSpec((_BLK, D), lambda i: (i, 0)),
                  pl.BlockSpec((D, D), lambda i: (0, 0))],
        out_specs=pl.BlockSpec((2, _BLK, DH), lambda i: (0, i, 0)),
        out_shape=jax.ShapeDtypeStruct((2, N_NODES, DH), jnp.float32),
    )(x, W)


def _dense2(hp, W):
    return pl.pallas_call(
        _dense2_body,
        grid=(N_NODES // _BLK,),
        in_specs=[pl.BlockSpec((2, _BLK, DH), lambda i: (0, i, 0)),
                  pl.BlockSpec((D, D), lambda i: (0, 0))],
        out_specs=pl.BlockSpec((2, _BLK, DH), lambda i: (0, i, 0)),
        out_shape=jax.ShapeDtypeStruct((2, N_NODES, DH), jnp.float32),
    )(hp, W)


def _concat(hp):
    return pl.pallas_call(
        _concat_body,
        grid=(N_NODES // _BLK,),
        in_specs=[pl.BlockSpec((2, _BLK, DH), lambda i: (0, i, 0))],
        out_specs=pl.BlockSpec((_BLK, D), lambda i: (i, 0)),
        out_shape=jax.ShapeDtypeStruct((N_NODES, D), jnp.float32),
    )(hp)


def _score(zh, a8t):
    return pl.pallas_call(
        _score_body,
        in_specs=[pl.BlockSpec((2, N_NODES, DH), lambda: (0, 0, 0)),
                  pl.BlockSpec((8, D), lambda: (0, 0))],
        out_specs=[pl.BlockSpec((8, N_NODES), lambda: (0, 0)),
                   pl.BlockSpec((8, 128), lambda: (0, 0))],
        out_shape=[jax.ShapeDtypeStruct((8, N_NODES), jnp.float32),
                   jax.ShapeDtypeStruct((8, 128), jnp.float32)],
    )(zh, a8t)


_MESH = plsc.VectorSubcoreMesh(core_axis_name="c", subcore_axis_name="s",
                               num_cores=NC, num_subcores=NS)


@functools.partial(
    pl.kernel,
    mesh=_MESH,
    out_type=jax.ShapeDtypeStruct((NC, N_NODES, DH), jnp.float32),
    compiler_params=pltpu.CompilerParams(needs_layout_passes=False,
                                         use_tc_tiling_on_sc=False),
    scratch_types=[
        pltpu.VMEM((N_NODES,), jnp.float32),
        pltpu.VMEM((N_NODES,), jnp.float32),
        pltpu.VMEM((6, 128), jnp.int32),
        pltpu.VMEM((6, 128), jnp.int32),
        pltpu.VMEM((2, 128), jnp.float32),
        pltpu.VMEM((4, 128, DH), jnp.float32),
        pltpu.VMEM((80,), jnp.float32),
        pltpu.VMEM((NN_T,), jnp.float32),
        pltpu.VMEM((16,), jnp.float32),
        pltpu.VMEM_SHARED((NPAD, DH), jnp.float32),
        pltpu.VMEM_SHARED((NPAD,), jnp.float32),
        pltpu.SemaphoreType.DMA,
        pltpu.SemaphoreType.DMA,
        pltpu.SemaphoreType.DMA,
        pltpu.SemaphoreType.DMA,
    ],
)
def _sc_layer(zl_hbm, zr_hbm, s8t_hbm, m_hbm, src_hbm, dst_hbm, hpart_hbm,
              s1v, s2v, srcd, dstd, exr, zbuf, dvm, dz, mv,
              acc_sh, den_sh, sem_g, sem_i, sem_d, sem_s):
    c = lax.axis_index("c")
    s = lax.axis_index("s")

    pltpu.sync_copy(s8t_hbm.at[0], s1v)
    pltpu.sync_copy(s8t_hbm.at[1], s2v)
    pltpu.sync_copy(m_hbm.at[0, pl.ds(0, 16)], mv)

    def _zrow(r, _):
        for k in range(DH // 16):
            zbuf[0, r, pl.ds(k * 16, 16)] = jnp.zeros((16,), jnp.float32)
        return 0
    lax.fori_loop(0, 128, _zrow, 0)

    def _zacc(k, _):
        pltpu.sync_copy(zbuf.at[0], acc_sh.at[pl.ds(s * NN_T + k * 128, 128)])
        return 0
    lax.fori_loop(0, NN_T // 128, _zacc, 0)

    def _zden(i, _):
        dz[pl.ds(i * 16, 16)] = jnp.zeros((16,), jnp.float32)
        return 0
    lax.fori_loop(0, NN_T // 16, _zden, 0)
    pltpu.sync_copy(dz, den_sh.at[pl.ds(s * NN_T, NN_T)])

    plsc.subcore_barrier()

    M = mv[...][0]
    row0 = s * RB

    def _fire_idx(j, slot):
        pltpu.async_copy(src_hbm.at[row0 + j], srcd.at[slot], sem_i)
        pltpu.async_copy(dst_hbm.at[row0 + j], dstd.at[slot], sem_i)

    def _wait_idx():
        pltpu.make_async_copy(src_hbm.at[0], srcd.at[0], sem_i).wait()
        pltpu.make_async_copy(src_hbm.at[0], srcd.at[0], sem_i).wait()

    def _edge_loop(ztab_hbm):
        _fire_idx(0, 0)
        _fire_idx(1, 1)
        _fire_idx(2, 2)
        _wait_idx()
        pltpu.async_copy(ztab_hbm.at[srcd.at[0]], zbuf.at[0], sem_g)
        _wait_idx()
        pltpu.async_copy(ztab_hbm.at[srcd.at[1]], zbuf.at[1], sem_g)

        def _body(j, _):
            buf = j % 4
            islot = j % 6
            eslot = j % 2

            @pl.when(j >= 2)
            def _():
                pltpu.make_async_copy(zbuf.at[0], acc_sh.at[dstd.at[0]],
                                      sem_s).wait()

            @pl.when(j < RB - 2)
            def _():
                _wait_idx()
                pltpu.async_copy(ztab_hbm.at[srcd.at[(j + 2) % 6]],
                                 zbuf.at[(j + 2) % 4], sem_g)

            @pl.when(j < RB - 3)
            def _():
                _fire_idx(j + 3, (j + 3) % 6)

            pltpu.make_async_copy(ztab_hbm.at[srcd.at[0]], zbuf.at[0],
                                  sem_g).wait()

            @pl.when(j >= 1)
            def _():
                pltpu.make_async_copy(exr.at[0], den_sh.at[dstd.at[0]],
                                      sem_d).wait()

            valid = row0 + j < VALID_ROWS

            for g in range(8):
                sv = srcd[islot, pl.ds(g * 16, 16)]
                dv = dstd[islot, pl.ds(g * 16, 16)]
                e = plsc.load_gather(s1v, [sv]) + plsc.load_gather(s2v, [dv])
                e = jnp.where(e > 0, e, LEAK * e) - M
                ex = jnp.exp(e)
                ex = jnp.where(valid, ex, jnp.zeros((16,), jnp.float32))
                exr[eslot, pl.ds(g * 16, 16)] = ex
                for t in range(16):
                    r = g * 16 + t
                    ex_s = ex[t]
                    for k in range(DH // 16):
                        zbuf[buf, r, pl.ds(k * 16, 16)] = (
                            zbuf[buf, r, pl.ds(k * 16, 16)] * ex_s)

            pltpu.async_copy(exr.at[eslot], den_sh.at[dstd.at[islot]], sem_d,
                             add=True)
            pltpu.async_copy(zbuf.at[buf], acc_sh.at[dstd.at[islot]], sem_s,
                             add=True)
            return 0
        lax.fori_loop(0, RB, _body, 0)
        pltpu.make_async_copy(zbuf.at[0], acc_sh.at[dstd.at[0]], sem_s).wait()
        pltpu.make_async_copy(zbuf.at[0], acc_sh.at[dstd.at[0]], sem_s).wait()
        pltpu.make_async_copy(exr.at[0], den_sh.at[dstd.at[0]], sem_d).wait()

    @pl.when(c == 0)
    def _():
        _edge_loop(zl_hbm)

    @pl.when(c == 1)
    def _():
        _edge_loop(zr_hbm)

    plsc.subcore_barrier()

    base0 = s * NN_T
    nch = jnp.where(s == NS - 1, (N_NODES - (NS - 1) * NN_T) // 80,
                    NN_T // 80)

    def _nbody(k, _):
        base = base0 + k * 80
        pltpu.sync_copy(acc_sh.at[pl.ds(base, 80)], zbuf.at[0, pl.ds(0, 80)])
        pltpu.sync_copy(den_sh.at[pl.ds(base, 80)], dvm)

        def _gbody(g, _):
            inv = 1.0 / (dvm[pl.ds(g * 16, 16)] + EPS)
            for t in range(16):
                r = g * 16 + t
                inv_s = inv[t]
                for k2 in range(DH // 16):
                    zbuf[0, r, pl.ds(k2 * 16, 16)] = (
                        zbuf[0, r, pl.ds(k2 * 16, 16)] * inv_s)
            return 0
        lax.fori_loop(0, 5, _gbody, 0)

        pltpu.sync_copy(zbuf.at[0, pl.ds(0, 80)],
                        hpart_hbm.at[c, pl.ds(base, 80)])
        return 0
    lax.fori_loop(0, nch, _nbody, 0)


def _att_vec(a):
    a8 = jnp.zeros((8, D), jnp.float32)
    a8 = a8.at[0].set(a[:D]).at[1].set(a[D:])
    return a8


def kernel(feature, edge_index, W1, a1, W2, a2):
    src = edge_index[0].astype(jnp.int32)
    dst = edge_index[1].astype(jnp.int32)
    src2d = jnp.pad(src, (0, EP - N_EDGES)).reshape(ROWS, 128)
    dst2d = jnp.pad(dst, (0, EP - N_EDGES)).reshape(ROWS, 128)

    zh1 = _dense1(feature, W1)
    s8t1, m1 = _score(zh1, _att_vec(a1))
    hp1 = _sc_layer(zh1[0], zh1[1], s8t1, m1, src2d, dst2d)
    zh2 = _dense2(hp1, W2)
    s8t2, m2 = _score(zh2, _att_vec(a2))
    hp2 = _sc_layer(zh2[0], zh2[1], s8t2, m2, src2d, dst2d)
    return _concat(hp2)
